# Initial kernel scaffold; baseline (speedup 1.0000x reference)
import jax
import jax.numpy as jnp
from jax.experimental import pallas as pl


def kernel(x, router_w, w_gate, w_up, w_down):
    raise NotImplementedError("write your pallas kernel here")


# speedup vs baseline: 1.6999x; 1.6999x over previous
import functools

import jax
import jax.numpy as jnp
from jax import lax
from jax.experimental import pallas as pl
from jax.experimental.pallas import tpu as pltpu
from jax.experimental.pallas import tpu_sc as plsc

HIDDEN = 1024
FF = 2816
E = 8
S = 2048
A = 2 * S
TILE = 256
NTILES = A // TILE
NSTEP = 32
NW = 32
F32 = jnp.float32
I32 = jnp.int32


def _eye(n):
    r = lax.broadcasted_iota(I32, (n, n), 0)
    c = lax.broadcasted_iota(I32, (n, n), 1)
    return (r == c).astype(F32)


def _transpose_row(row, n):
    return lax.dot_general(_eye(n), row, (((1,), (1,)), ((), ())),
                           precision=lax.Precision.HIGHEST,
                           preferred_element_type=F32)


def _router_body(x_ref, rw_ref, pos0_ref, pos1_ref, w0_ref, w1_ref,
                 tile_ref, exp_ref, rs_ref, re_ref, init_ref, loss_ref):
    x = x_ref[...]
    rw = rw_ref[...]
    logits = lax.dot_general(x, rw, (((1,), (1,)), ((), ())),
                             preferred_element_type=F32)
    m = jnp.max(logits, axis=-1, keepdims=True)
    ex = jnp.exp(logits - m)
    probs = ex / jnp.sum(ex, axis=-1, keepdims=True)

    idx8 = lax.broadcasted_iota(I32, (S, E), 1)
    p0 = jnp.max(probs, axis=-1, keepdims=True)
    i0 = jnp.min(jnp.where(probs == p0, idx8, E), axis=-1, keepdims=True)
    h0 = (idx8 == i0).astype(F32)
    masked = jnp.where(h0 > 0, -1.0, probs)
    p1 = jnp.max(masked, axis=-1, keepdims=True)
    i1 = jnp.min(jnp.where(masked == p1, idx8, E), axis=-1, keepdims=True)
    h1 = (idx8 == i1).astype(F32)

    wsum = p0 + p1
    w0_ref[...] = p0 / wsum
    w1_ref[...] = p1 / wsum

    avg = jnp.sum(probs, axis=0, keepdims=True) * (1.0 / S)
    frac = jnp.sum((probs > 0).astype(F32), axis=0, keepdims=True) * (1.0 / S)
    loss_ref[...] = jnp.sum(avg * frac, keepdims=True).reshape(1, 1) * float(E)

    c0 = jnp.sum(h0, axis=0, keepdims=True)
    c1 = jnp.sum(h1, axis=0, keepdims=True)
    counts = c0 + c1
    r8 = lax.broadcasted_iota(I32, (E, E), 0)
    col8 = lax.broadcasted_iota(I32, (E, E), 1)
    tri = (r8 < col8).astype(F32)
    offs = lax.dot_general(counts, tri, (((1,), (0,)), ((), ())),
                           precision=lax.Precision.HIGHEST,
                           preferred_element_type=F32)

    h01 = jnp.concatenate([h0, h1], axis=1)
    chunks = []
    csz = 512
    for cstart in range(0, S, csz):
        rr = lax.broadcasted_iota(I32, (csz, S), 0) + cstart
        cc = lax.broadcasted_iota(I32, (csz, S), 1)
        lex = (cc < rr).astype(F32)
        chunks.append(lax.dot_general(lex, h01, (((1,), (0,)), ((), ())),
                                      precision=lax.Precision.HIGHEST,
                                      preferred_element_type=F32))
    c01 = jnp.concatenate(chunks, axis=0)

    pos0f = jnp.sum(h0 * (offs + c01[:, :E]), axis=1, keepdims=True)
    pos1f = jnp.sum(h1 * (offs + c0 + c01[:, E:]), axis=1, keepdims=True)
    pos0_ref[...] = pos0f.astype(I32)
    pos1_ref[...] = pos1f.astype(I32)

    lane = lax.broadcasted_iota(I32, (1, NSTEP), 1)
    lanef = lane.astype(F32)
    re_sel = lax.broadcasted_iota(I32, (E, NSTEP), 0)
    cl_sel = lax.broadcasted_iota(I32, (E, NSTEP), 1)
    sel = ((re_sel >= 1) & (cl_sel == re_sel + (NTILES - 1))).astype(F32)
    off_bc = lax.dot_general(offs, sel, (((1,), (0,)), ((), ())),
                             precision=lax.Precision.HIGHEST,
                             preferred_element_type=F32)
    cand = jnp.where(lane < NTILES, lanef * TILE,
                     jnp.where(lane < NTILES + E - 1, off_bc, float(A)))
    cand_c = _transpose_row(cand, NSTEP)
    sub = lax.broadcasted_iota(I32, (NSTEP, 1), 0)
    lt = (cand < cand_c).astype(F32)
    eqlt = ((cand == cand_c) & (lane < sub)).astype(F32)
    rank = jnp.sum(lt + eqlt, axis=1, keepdims=True)
    r_is = (rank == lanef).astype(F32)
    ss = jnp.sum(r_is * cand_c, axis=0, keepdims=True)
    r_next = (rank == lanef + 1.0).astype(F32)
    ee = jnp.sum(r_next * cand_c, axis=0, keepdims=True)
    ee = jnp.maximum(ee, ss)
    ssi = ss.astype(I32)
    eei = ee.astype(I32)
    tile_i = jnp.minimum(ssi // TILE, NTILES - 1)
    offs_c = _transpose_row(offs, E)
    exp_f = jnp.sum((offs_c <= ss).astype(F32), axis=0, keepdims=True) - 1.0
    exp_ref[...] = jnp.clip(exp_f.astype(I32), 0, E - 1)
    rs_ref[...] = ssi - tile_i * TILE
    re_ref[...] = eei - tile_i * TILE
    tile_ref[...] = tile_i
    tile_f = tile_i.astype(F32)
    rl = lax.broadcasted_iota(I32, (NSTEP, NSTEP), 0)
    cs = lax.broadcasted_iota(I32, (NSTEP, NSTEP), 1)
    shift = (rl + 1 == cs).astype(F32)
    tile_prev = lax.dot_general(tile_f, shift, (((1,), (0,)), ((), ())),
                                precision=lax.Precision.HIGHEST,
                                preferred_element_type=F32)
    init_ref[...] = ((lane == 0) | (tile_f != tile_prev)).astype(I32)


def _router(xf, router_w):
    outs = (
        jax.ShapeDtypeStruct((S, 1), I32),
        jax.ShapeDtypeStruct((S, 1), I32),
        jax.ShapeDtypeStruct((S, 1), F32),
        jax.ShapeDtypeStruct((S, 1), F32),
        jax.ShapeDtypeStruct((1, NSTEP), I32),
        jax.ShapeDtypeStruct((1, NSTEP), I32),
        jax.ShapeDtypeStruct((1, NSTEP), I32),
        jax.ShapeDtypeStruct((1, NSTEP), I32),
        jax.ShapeDtypeStruct((1, NSTEP), I32),
        jax.ShapeDtypeStruct((1, 1), F32),
    )
    return pl.pallas_call(_router_body, out_shape=outs)(xf, router_w)


def _wscatter_body(pos0_ref, pos1_ref, w0_ref, w1_ref, out_ref):
    r = pl.program_id(0)
    csz = 512
    lane = lax.broadcasted_iota(I32, (1, csz), 1) + r * csz
    lanef = lane.astype(F32)
    m0 = (pos0_ref[...].astype(F32) == lanef).astype(F32)
    m1 = (pos1_ref[...].astype(F32) == lanef).astype(F32)
    acc = (jnp.sum(m0 * w0_ref[...], axis=0, keepdims=True) +
           jnp.sum(m1 * w1_ref[...], axis=0, keepdims=True))
    out_ref[...] = acc.reshape(1, 1, csz)


def _wscatter(pos0, pos1, w0, w1):
    csz = 512
    grid = A // csz
    spec_full = pl.BlockSpec((S, 1), lambda r: (0, 0))
    return pl.pallas_call(
        _wscatter_body,
        grid=(grid,),
        in_specs=[spec_full, spec_full, spec_full, spec_full],
        out_specs=pl.BlockSpec((1, 1, csz), lambda r: (r, 0, 0)),
        out_shape=jax.ShapeDtypeStruct((grid, 1, csz), F32),
    )(pos0, pos1, w0, w1)


def _dispatch_body(x_hbm, p0_hbm, p1_hbm, out_hbm, i0_v, i1_v, rows_v, sem):
    wid = lax.axis_index("s") * 2 + lax.axis_index("c")
    chunk = S // NW
    base = wid * chunk
    pltpu.sync_copy(p0_hbm.at[pl.ds(base, chunk)], i0_v)
    pltpu.sync_copy(p1_hbm.at[pl.ds(base, chunk)], i1_v)
    pltpu.sync_copy(x_hbm.at[pl.ds(base, chunk), :], rows_v)
    pltpu.async_copy(rows_v, out_hbm.at[i0_v], sem).wait()
    pltpu.async_copy(rows_v, out_hbm.at[i1_v], sem).wait()


def _dispatch(xf, pos0, pos1):
    chunk = S // NW
    mesh = plsc.VectorSubcoreMesh(core_axis_name="c", subcore_axis_name="s")
    return pl.kernel(
        _dispatch_body,
        out_type=jax.ShapeDtypeStruct((A, HIDDEN), F32),
        mesh=mesh,
        scratch_types=[
            pltpu.VMEM((chunk,), I32),
            pltpu.VMEM((chunk,), I32),
            pltpu.VMEM((chunk, HIDDEN), F32),
            pltpu.SemaphoreType.DMA,
        ],
    )(xf, pos0, pos1)


def _grouped_body(tile_r, exp_r, rs_r, re_r, init_r,
                  xs_ref, ws_ref, wg_ref, wu_ref, wd_ref, ys_ref):
    s = pl.program_id(0)
    rs = rs_r[s]
    re = re_r[s]
    init = init_r[s]
    rowi = lax.broadcasted_iota(I32, (TILE, 1), 0)
    msk = ((rowi >= rs) & (rowi < re)).astype(F32)
    xb = (xs_ref[...] * msk).astype(jnp.bfloat16)
    g = lax.dot_general(xb, wg_ref[0], (((1,), (1,)), ((), ())),
                        preferred_element_type=F32)
    u = lax.dot_general(xb, wu_ref[0], (((1,), (1,)), ((), ())),
                        preferred_element_type=F32)
    h = (g * jax.nn.sigmoid(g) * u).astype(jnp.bfloat16)
    y = lax.dot_general(h, wd_ref[0], (((1,), (1,)), ((), ())),
                        preferred_element_type=F32)
    y = y * ws_ref[...]

    @pl.when(init == 1)
    def _():
        ys_ref[...] = y

    @pl.when(init == 0)
    def _():
        ys_ref[...] += y


def _grouped(meta, xs, ws2, w_gate, w_up, w_down):
    grid_spec = pltpu.PrefetchScalarGridSpec(
        num_scalar_prefetch=5,
        grid=(NSTEP,),
        in_specs=[
            pl.BlockSpec((TILE, HIDDEN), lambda s, t, e, a, b, i: (t[s], 0)),
            pl.BlockSpec((TILE, 1), lambda s, t, e, a, b, i: (t[s], 0)),
            pl.BlockSpec((1, FF, HIDDEN), lambda s, t, e, a, b, i: (e[s], 0, 0)),
            pl.BlockSpec((1, FF, HIDDEN), lambda s, t, e, a, b, i: (e[s], 0, 0)),
            pl.BlockSpec((1, HIDDEN, FF), lambda s, t, e, a, b, i: (e[s], 0, 0)),
        ],
        out_specs=pl.BlockSpec((TILE, HIDDEN), lambda s, t, e, a, b, i: (t[s], 0)),
    )
    return pl.pallas_call(
        _grouped_body,
        grid_spec=grid_spec,
        out_shape=jax.ShapeDtypeStruct((A, HIDDEN), F32),
    )(*meta, xs, ws2, w_gate, w_up, w_down)


def _combine_body(ys_hbm, p0_hbm, p1_hbm, out_hbm, i0_v, i1_v, b0, b1, sem):
    wid = lax.axis_index("s") * 2 + lax.axis_index("c")
    chunk = 32
    nsub = S // (NW * chunk)
    for sub in range(nsub):
        base = wid * (chunk * nsub) + sub * chunk
        pltpu.sync_copy(p0_hbm.at[pl.ds(base, chunk)], i0_v)
        pltpu.sync_copy(p1_hbm.at[pl.ds(base, chunk)], i1_v)
        pltpu.async_copy(ys_hbm.at[i0_v], b0, sem).wait()
        pltpu.async_copy(ys_hbm.at[i1_v], b1, sem).wait()

        def _row(i, carry):
            def _col(j, carry2):
                sl = pl.ds(j * 16, 16)
                b0[i, sl] = b0[i, sl] + b1[i, sl]
                return carry2
            return lax.fori_loop(0, HIDDEN // 16, _col, carry)

        lax.fori_loop(0, chunk, _row, 0)
        pltpu.sync_copy(b0, out_hbm.at[pl.ds(base, chunk), :])


def _combine(ys, pos0, pos1):
    chunk = 32
    mesh = plsc.VectorSubcoreMesh(core_axis_name="c", subcore_axis_name="s")
    return pl.kernel(
        _combine_body,
        out_type=jax.ShapeDtypeStruct((S, HIDDEN), F32),
        mesh=mesh,
        scratch_types=[
            pltpu.VMEM((chunk,), I32),
            pltpu.VMEM((chunk,), I32),
            pltpu.VMEM((chunk, HIDDEN), F32),
            pltpu.VMEM((chunk, HIDDEN), F32),
            pltpu.SemaphoreType.DMA,
        ],
    )(ys, pos0, pos1)


@jax.jit
def kernel(x, router_w, w_gate, w_up, w_down):
    bsz, seq, dim = x.shape
    xf = x.reshape(seq, dim)
    (pos0, pos1, w0n, w1n, tile_a, exp_a, rs_a, re_a, init_a,
     loss) = _router(xf, router_w)
    wsort = _wscatter(pos0, pos1, w0n, w1n).reshape(A, 1)
    p0v = pos0.reshape(S)
    p1v = pos1.reshape(S)
    xs = _dispatch(xf, p0v, p1v)
    meta = (tile_a.reshape(NSTEP), exp_a.reshape(NSTEP), rs_a.reshape(NSTEP),
            re_a.reshape(NSTEP), init_a.reshape(NSTEP))
    bf16 = jnp.bfloat16
    ys = _grouped(meta, xs, wsort, w_gate.astype(bf16), w_up.astype(bf16),
                  w_down.astype(bf16))
    out = _combine(ys, p0v, p1v)
    return out.reshape(bsz, seq, dim), loss.reshape(())


# speedup vs baseline: 1.9620x; 1.1542x over previous
import functools

import jax
import jax.numpy as jnp
from jax import lax
from jax.experimental import pallas as pl
from jax.experimental.pallas import tpu as pltpu
from jax.experimental.pallas import tpu_sc as plsc

HIDDEN = 1024
FF = 2816
E = 8
S = 2048
A = 2 * S
TILE = 256
NTILES = A // TILE
NSTEP = 32
NW = 32
F32 = jnp.float32
I32 = jnp.int32


def _eye(n):
    r = lax.broadcasted_iota(I32, (n, n), 0)
    c = lax.broadcasted_iota(I32, (n, n), 1)
    return (r == c).astype(F32)


def _transpose_row(row, n):
    return lax.dot_general(_eye(n), row, (((1,), (1,)), ((), ())),
                           precision=lax.Precision.HIGHEST,
                           preferred_element_type=F32)


def _router_body(x_ref, rw_ref, pos0_ref, pos1_ref, w0_ref, w1_ref,
                 tile_ref, exp_ref, rs_ref, re_ref, init_ref, loss_ref):
    x = x_ref[...]
    rw = rw_ref[...]
    logits = lax.dot_general(x, rw, (((1,), (1,)), ((), ())),
                             preferred_element_type=F32)
    m = jnp.max(logits, axis=-1, keepdims=True)
    ex = jnp.exp(logits - m)
    probs = ex / jnp.sum(ex, axis=-1, keepdims=True)

    idx8 = lax.broadcasted_iota(I32, (S, E), 1)
    p0 = jnp.max(probs, axis=-1, keepdims=True)
    i0 = jnp.min(jnp.where(probs == p0, idx8, E), axis=-1, keepdims=True)
    h0 = (idx8 == i0).astype(F32)
    masked = jnp.where(h0 > 0, -1.0, probs)
    p1 = jnp.max(masked, axis=-1, keepdims=True)
    i1 = jnp.min(jnp.where(masked == p1, idx8, E), axis=-1, keepdims=True)
    h1 = (idx8 == i1).astype(F32)

    wsum = p0 + p1
    w0_ref[...] = p0 / wsum
    w1_ref[...] = p1 / wsum

    avg = jnp.sum(probs, axis=0, keepdims=True) * (1.0 / S)
    frac = jnp.sum((probs > 0).astype(F32), axis=0, keepdims=True) * (1.0 / S)
    loss_ref[...] = jnp.sum(avg * frac, keepdims=True).reshape(1, 1) * float(E)

    c0 = jnp.sum(h0, axis=0, keepdims=True)
    c1 = jnp.sum(h1, axis=0, keepdims=True)
    counts = c0 + c1
    r8 = lax.broadcasted_iota(I32, (E, E), 0)
    col8 = lax.broadcasted_iota(I32, (E, E), 1)
    tri = (r8 < col8).astype(F32)
    offs = lax.dot_general(counts, tri, (((1,), (0,)), ((), ())),
                           precision=lax.Precision.HIGHEST,
                           preferred_element_type=F32)

    h01 = jnp.concatenate([h0, h1], axis=1)
    chunks = []
    csz = 512
    for cstart in range(0, S, csz):
        rr = lax.broadcasted_iota(I32, (csz, S), 0) + cstart
        cc = lax.broadcasted_iota(I32, (csz, S), 1)
        lex = (cc < rr).astype(F32)
        chunks.append(lax.dot_general(lex, h01, (((1,), (0,)), ((), ())),
                                      precision=lax.Precision.HIGHEST,
                                      preferred_element_type=F32))
    c01 = jnp.concatenate(chunks, axis=0)

    pos0f = jnp.sum(h0 * (offs + c01[:, :E]), axis=1, keepdims=True)
    pos1f = jnp.sum(h1 * (offs + c0 + c01[:, E:]), axis=1, keepdims=True)
    pos0_ref[...] = pos0f.astype(I32)
    pos1_ref[...] = pos1f.astype(I32)

    lane = lax.broadcasted_iota(I32, (1, NSTEP), 1)
    lanef = lane.astype(F32)
    re_sel = lax.broadcasted_iota(I32, (E, NSTEP), 0)
    cl_sel = lax.broadcasted_iota(I32, (E, NSTEP), 1)
    sel = ((re_sel >= 1) & (cl_sel == re_sel + (NTILES - 1))).astype(F32)
    off_bc = lax.dot_general(offs, sel, (((1,), (0,)), ((), ())),
                             precision=lax.Precision.HIGHEST,
                             preferred_element_type=F32)
    cand = jnp.where(lane < NTILES, lanef * TILE,
                     jnp.where(lane < NTILES + E - 1, off_bc, float(A)))
    cand_c = _transpose_row(cand, NSTEP)
    sub = lax.broadcasted_iota(I32, (NSTEP, 1), 0)
    lt = (cand < cand_c).astype(F32)
    eqlt = ((cand == cand_c) & (lane < sub)).astype(F32)
    rank = jnp.sum(lt + eqlt, axis=1, keepdims=True)
    r_is = (rank == lanef).astype(F32)
    ss = jnp.sum(r_is * cand_c, axis=0, keepdims=True)
    r_next = (rank == lanef + 1.0).astype(F32)
    ee = jnp.sum(r_next * cand_c, axis=0, keepdims=True)
    ee = jnp.maximum(ee, ss)
    ssi = ss.astype(I32)
    eei = ee.astype(I32)
    tile_i = jnp.minimum(ssi // TILE, NTILES - 1)
    offs_c = _transpose_row(offs, E)
    exp_f = jnp.sum((offs_c <= ss).astype(F32), axis=0, keepdims=True) - 1.0
    exp_ref[...] = jnp.clip(exp_f.astype(I32), 0, E - 1)
    rs_ref[...] = ssi - tile_i * TILE
    re_ref[...] = eei - tile_i * TILE
    tile_ref[...] = tile_i
    tile_f = tile_i.astype(F32)
    rl = lax.broadcasted_iota(I32, (NSTEP, NSTEP), 0)
    cs = lax.broadcasted_iota(I32, (NSTEP, NSTEP), 1)
    shift = (rl + 1 == cs).astype(F32)
    tile_prev = lax.dot_general(tile_f, shift, (((1,), (0,)), ((), ())),
                                precision=lax.Precision.HIGHEST,
                                preferred_element_type=F32)
    init_ref[...] = ((lane == 0) | (tile_f != tile_prev)).astype(I32)


def _router(xf, router_w):
    outs = (
        jax.ShapeDtypeStruct((S, 1), I32),
        jax.ShapeDtypeStruct((S, 1), I32),
        jax.ShapeDtypeStruct((S, 1), F32),
        jax.ShapeDtypeStruct((S, 1), F32),
        jax.ShapeDtypeStruct((1, NSTEP), I32),
        jax.ShapeDtypeStruct((1, NSTEP), I32),
        jax.ShapeDtypeStruct((1, NSTEP), I32),
        jax.ShapeDtypeStruct((1, NSTEP), I32),
        jax.ShapeDtypeStruct((1, NSTEP), I32),
        jax.ShapeDtypeStruct((1, 1), F32),
    )
    return pl.pallas_call(_router_body, out_shape=outs)(xf, router_w)


def _wscatter_body(pos0_ref, pos1_ref, w0_ref, w1_ref, out_ref):
    r = pl.program_id(0)
    csz = 512
    lane = lax.broadcasted_iota(I32, (1, csz), 1) + r * csz
    lanef = lane.astype(F32)
    m0 = (pos0_ref[...].astype(F32) == lanef).astype(F32)
    m1 = (pos1_ref[...].astype(F32) == lanef).astype(F32)
    acc = (jnp.sum(m0 * w0_ref[...], axis=0, keepdims=True) +
           jnp.sum(m1 * w1_ref[...], axis=0, keepdims=True))
    out_ref[...] = acc.reshape(1, 1, csz)


def _wscatter(pos0, pos1, w0, w1):
    csz = 512
    grid = A // csz
    spec_full = pl.BlockSpec((S, 1), lambda r: (0, 0))
    return pl.pallas_call(
        _wscatter_body,
        grid=(grid,),
        in_specs=[spec_full, spec_full, spec_full, spec_full],
        out_specs=pl.BlockSpec((1, 1, csz), lambda r: (r, 0, 0)),
        out_shape=jax.ShapeDtypeStruct((grid, 1, csz), F32),
    )(pos0, pos1, w0, w1)


def _dispatch_body(x_hbm, p0_hbm, p1_hbm, out_hbm, i0_v, i1_v, rows_v, sem):
    wid = lax.axis_index("s") * 2 + lax.axis_index("c")
    chunk = S // NW
    base = wid * chunk
    pltpu.sync_copy(p0_hbm.at[pl.ds(base, chunk)], i0_v)
    pltpu.sync_copy(p1_hbm.at[pl.ds(base, chunk)], i1_v)
    pltpu.sync_copy(x_hbm.at[pl.ds(base, chunk), :], rows_v)
    pltpu.async_copy(rows_v, out_hbm.at[i0_v], sem).wait()
    pltpu.async_copy(rows_v, out_hbm.at[i1_v], sem).wait()


def _dispatch(xf, pos0, pos1):
    chunk = S // NW
    mesh = plsc.VectorSubcoreMesh(core_axis_name="c", subcore_axis_name="s")
    return pl.kernel(
        _dispatch_body,
        out_type=jax.ShapeDtypeStruct((A, HIDDEN), F32),
        mesh=mesh,
        scratch_types=[
            pltpu.VMEM((chunk,), I32),
            pltpu.VMEM((chunk,), I32),
            pltpu.VMEM((chunk, HIDDEN), F32),
            pltpu.SemaphoreType.DMA,
        ],
    )(xf, pos0, pos1)


def _grouped_body(tile_r, exp_r, rs_r, re_r, init_r,
                  xs_ref, ws_ref, wg_ref, wu_ref, wd_ref, ys_ref):
    s = pl.program_id(0)
    rs = rs_r[s]
    re = re_r[s]
    init = init_r[s]
    rowi = lax.broadcasted_iota(I32, (TILE, 1), 0)
    msk = ((rowi >= rs) & (rowi < re)).astype(F32)
    xb = xs_ref[...] * msk
    g = lax.dot_general(xb, wg_ref[0], (((1,), (1,)), ((), ())),
                        preferred_element_type=F32)
    u = lax.dot_general(xb, wu_ref[0], (((1,), (1,)), ((), ())),
                        preferred_element_type=F32)
    h = g * jax.nn.sigmoid(g) * u
    y = lax.dot_general(h, wd_ref[0], (((1,), (1,)), ((), ())),
                        preferred_element_type=F32)
    y = y * ws_ref[...]

    @pl.when(init == 1)
    def _():
        ys_ref[...] = y

    @pl.when(init == 0)
    def _():
        ys_ref[...] += y


def _grouped(meta, xs, ws2, w_gate, w_up, w_down):
    grid_spec = pltpu.PrefetchScalarGridSpec(
        num_scalar_prefetch=5,
        grid=(NSTEP,),
        in_specs=[
            pl.BlockSpec((TILE, HIDDEN), lambda s, t, e, a, b, i: (t[s], 0)),
            pl.BlockSpec((TILE, 1), lambda s, t, e, a, b, i: (t[s], 0)),
            pl.BlockSpec((1, FF, HIDDEN), lambda s, t, e, a, b, i: (e[s], 0, 0),
                         pipeline_mode=pl.Buffered(buffer_count=1)),
            pl.BlockSpec((1, FF, HIDDEN), lambda s, t, e, a, b, i: (e[s], 0, 0),
                         pipeline_mode=pl.Buffered(buffer_count=1)),
            pl.BlockSpec((1, HIDDEN, FF), lambda s, t, e, a, b, i: (e[s], 0, 0),
                         pipeline_mode=pl.Buffered(buffer_count=1)),
        ],
        out_specs=pl.BlockSpec((TILE, HIDDEN), lambda s, t, e, a, b, i: (t[s], 0)),
    )
    return pl.pallas_call(
        _grouped_body,
        grid_spec=grid_spec,
        out_shape=jax.ShapeDtypeStruct((A, HIDDEN), F32),
    )(*meta, xs, ws2, w_gate, w_up, w_down)


def _combine_body(ys_hbm, p0_hbm, p1_hbm, out_hbm, i0_v, i1_v, b0, b1, sem):
    wid = lax.axis_index("s") * 2 + lax.axis_index("c")
    chunk = 32
    nsub = S // (NW * chunk)
    for sub in range(nsub):
        base = wid * (chunk * nsub) + sub * chunk
        pltpu.sync_copy(p0_hbm.at[pl.ds(base, chunk)], i0_v)
        pltpu.sync_copy(p1_hbm.at[pl.ds(base, chunk)], i1_v)
        pltpu.async_copy(ys_hbm.at[i0_v], b0, sem).wait()
        pltpu.async_copy(ys_hbm.at[i1_v], b1, sem).wait()

        def _row(i, carry):
            def _col(j, carry2):
                sl = pl.ds(j * 16, 16)
                b0[i, sl] = b0[i, sl] + b1[i, sl]
                return carry2
            return lax.fori_loop(0, HIDDEN // 16, _col, carry)

        lax.fori_loop(0, chunk, _row, 0)
        pltpu.sync_copy(b0, out_hbm.at[pl.ds(base, chunk), :])


def _combine(ys, pos0, pos1):
    chunk = 32
    mesh = plsc.VectorSubcoreMesh(core_axis_name="c", subcore_axis_name="s")
    return pl.kernel(
        _combine_body,
        out_type=jax.ShapeDtypeStruct((S, HIDDEN), F32),
        mesh=mesh,
        scratch_types=[
            pltpu.VMEM((chunk,), I32),
            pltpu.VMEM((chunk,), I32),
            pltpu.VMEM((chunk, HIDDEN), F32),
            pltpu.VMEM((chunk, HIDDEN), F32),
            pltpu.SemaphoreType.DMA,
        ],
    )(ys, pos0, pos1)


@jax.jit
def kernel(x, router_w, w_gate, w_up, w_down):
    bsz, seq, dim = x.shape
    xf = x.reshape(seq, dim)
    (pos0, pos1, w0n, w1n, tile_a, exp_a, rs_a, re_a, init_a,
     loss) = _router(xf, router_w)
    wsort = _wscatter(pos0, pos1, w0n, w1n).reshape(A, 1)
    p0v = pos0.reshape(S)
    p1v = pos1.reshape(S)
    xs = _dispatch(xf, p0v, p1v)
    meta = (tile_a.reshape(NSTEP), exp_a.reshape(NSTEP), rs_a.reshape(NSTEP),
            re_a.reshape(NSTEP), init_a.reshape(NSTEP))
    ys = _grouped(meta, xs, wsort, w_gate, w_up, w_down)
    out = _combine(ys, p0v, p1v)
    return out.reshape(bsz, seq, dim), loss.reshape(())
